# baseline (speedup 1.0000x reference)
import functools

import jax
import jax.numpy as jnp
from jax import lax
from jax.experimental import pallas as pl
from jax.experimental.pallas import tpu as pltpu
from jax.experimental.pallas import tpu_sc as plsc


def kernel(user_id, item_id, diff, user_table, item_table, W_out, b_out):
    B = user_id.shape[0]
    D = user_table.shape[1]
    info = plsc.get_sparse_core_info()
    NC, NS, L = info.num_cores, info.num_subcores, info.num_lanes
    NW = NC * NS
    b_per_w = B // NW
    CH = 128
    n_ch = b_per_w // CH
    n_grp = b_per_w // L

    uid3 = user_id.reshape(NW, n_ch, CH)
    iid3 = item_id.reshape(NW, n_ch, CH)
    w16 = jnp.broadcast_to(W_out.reshape(1), (L,))
    b16 = jnp.broadcast_to(b_out, (L,))

    mesh = plsc.VectorSubcoreMesh(core_axis_name="c", subcore_axis_name="s")

    @functools.partial(
        pl.kernel,
        mesh=mesh,
        out_type=jax.ShapeDtypeStruct((B,), jnp.float32),
        scratch_types=[
            pltpu.VMEM((n_ch, CH), jnp.int32),
            pltpu.VMEM((n_ch, CH), jnp.int32),
            pltpu.VMEM((b_per_w, D), jnp.float32),
            pltpu.VMEM((b_per_w, D), jnp.float32),
            pltpu.VMEM((b_per_w,), jnp.float32),
            pltpu.VMEM((b_per_w,), jnp.float32),
            pltpu.VMEM((L,), jnp.float32),
            pltpu.VMEM((L,), jnp.float32),
            pltpu.SemaphoreType.DMA,
        ],
        compiler_params=pltpu.CompilerParams(
            needs_layout_passes=False, use_tc_tiling_on_sc=False),
    )
    def mf_kernel(uid_h, iid_h, diff_h, ut_h, it_h, w_h, b_h, out_h,
                  uidx_v, iidx_v, urows_v, irows_v, diff_v, out_v, w_v, b_v,
                  sem):
        wid = lax.axis_index("s") * NC + lax.axis_index("c")
        base = wid * b_per_w

        pltpu.sync_copy(uid_h.at[wid], uidx_v)
        pltpu.sync_copy(iid_h.at[wid], iidx_v)
        pltpu.sync_copy(diff_h.at[pl.ds(base, b_per_w)], diff_v)
        pltpu.sync_copy(w_h, w_v)
        pltpu.sync_copy(b_h, b_v)

        copies = []
        for j in range(n_ch):
            copies.append(pltpu.async_copy(
                ut_h.at[uidx_v.at[j]], urows_v.at[pl.ds(j * CH, CH)], sem))
            copies.append(pltpu.async_copy(
                it_h.at[iidx_v.at[j]], irows_v.at[pl.ds(j * CH, CH)], sem))
        for c in copies:
            c.wait()

        w = w_v[...]
        b = b_v[...]
        riota = lax.iota(jnp.int32, L)

        def body(g, carry):
            rows = riota + g * L
            acc = jnp.zeros((L,), jnp.float32)
            for d in range(D):
                dcol = jnp.full((L,), d, jnp.int32)
                uc = plsc.load_gather(urows_v, [rows, dcol])
                ic = plsc.load_gather(irows_v, [rows, dcol])
                acc = acc + uc * ic
            sig = 1.0 / (1.0 + jnp.exp(-acc))
            dv = diff_v[pl.ds(g * L, L)]
            y = sig * dv * w + b
            out_v[pl.ds(g * L, L)] = 1.0 / (1.0 + jnp.exp(-y))
            return carry

        lax.fori_loop(0, n_grp, body, 0)
        pltpu.sync_copy(out_v, out_h.at[pl.ds(base, b_per_w)])

    return mf_kernel(uid3, iid3, diff, user_table, item_table, w16, b16)
